# baseline (speedup 1.0000x reference)
import jax
import jax.numpy as jnp
from jax.experimental import pallas as pl
from jax.experimental.pallas import tpu as pltpu

_BR = 200


def _support_kernel(x_ref, w_ref, s_ref):
    s_ref[...] = jnp.dot(x_ref[...], w_ref[...], preferred_element_type=jnp.float32)


def _spmm_kernel(s_ref, b_ref, adj_ref, out_ref):
    out_ref[...] = (
        jnp.dot(adj_ref[...], s_ref[...], preferred_element_type=jnp.float32)
        + b_ref[...]
    )


def kernel(input, adj, weight, bias):
    n, f_in = input.shape
    f_out = weight.shape[1]
    bias2d = bias.reshape(1, f_out)

    support = pl.pallas_call(
        _support_kernel,
        out_shape=jax.ShapeDtypeStruct((n, f_out), jnp.float32),
    )(input, weight)

    return pl.pallas_call(
        _spmm_kernel,
        grid=(adj.shape[0] // _BR,),
        in_specs=[
            pl.BlockSpec((n, f_out), lambda i: (0, 0)),
            pl.BlockSpec((1, f_out), lambda i: (0, 0)),
            pl.BlockSpec((_BR, n), lambda i: (i, 0)),
        ],
        out_specs=pl.BlockSpec((_BR, f_out), lambda i: (i, 0)),
        out_shape=jax.ShapeDtypeStruct((adj.shape[0], f_out), jnp.float32),
        compiler_params=pltpu.CompilerParams(dimension_semantics=("parallel",)),
    )(support, bias2d, adj)


# speedup vs baseline: 1.0306x; 1.0306x over previous
import jax
import jax.numpy as jnp
from jax.experimental import pallas as pl
from jax.experimental.pallas import tpu as pltpu

_BR = 400


def _gc_kernel(x_ref, w_ref, b_ref, adj_ref, out_ref, support_ref):
    @pl.when(pl.program_id(0) == 0)
    def _():
        support_ref[...] = jnp.dot(
            x_ref[...], w_ref[...], preferred_element_type=jnp.float32
        )

    out_ref[...] = (
        jnp.dot(adj_ref[...], support_ref[...], preferred_element_type=jnp.float32)
        + b_ref[...]
    )


def kernel(input, adj, weight, bias):
    n, f_in = input.shape
    f_out = weight.shape[1]
    bias2d = bias.reshape(1, f_out)
    grid = (adj.shape[0] // _BR,)
    return pl.pallas_call(
        _gc_kernel,
        grid=grid,
        in_specs=[
            pl.BlockSpec((n, f_in), lambda i: (0, 0)),
            pl.BlockSpec((f_in, f_out), lambda i: (0, 0)),
            pl.BlockSpec((1, f_out), lambda i: (0, 0)),
            pl.BlockSpec((_BR, n), lambda i: (i, 0)),
        ],
        out_specs=pl.BlockSpec((_BR, f_out), lambda i: (i, 0)),
        out_shape=jax.ShapeDtypeStruct((adj.shape[0], f_out), jnp.float32),
        scratch_shapes=[pltpu.VMEM((n, f_out), jnp.float32)],
    )(input, weight, bias2d, adj)


# speedup vs baseline: 1.0387x; 1.0079x over previous
import jax
import jax.numpy as jnp
from jax.experimental import pallas as pl
from jax.experimental.pallas import tpu as pltpu

_BR = 400
_GROUP = 5


def _gc_kernel(x_ref, w_ref, b_ref, adj_ref, out_ref, support_ref):
    i = pl.program_id(0)
    k = pl.program_id(1)

    @pl.when((i == 0) & (k == 0))
    def _():
        support_ref[...] = jnp.dot(
            x_ref[...], w_ref[...], preferred_element_type=jnp.float32
        )

    out_ref[pl.ds(k * _BR, _BR), :] = (
        jnp.dot(adj_ref[...], support_ref[...], preferred_element_type=jnp.float32)
        + b_ref[...]
    )


def kernel(input, adj, weight, bias):
    n, f_in = input.shape
    f_out = weight.shape[1]
    bias2d = bias.reshape(1, f_out)
    grid = (adj.shape[0] // (_BR * _GROUP), _GROUP)
    return pl.pallas_call(
        _gc_kernel,
        grid=grid,
        in_specs=[
            pl.BlockSpec((n, f_in), lambda i, k: (0, 0)),
            pl.BlockSpec((f_in, f_out), lambda i, k: (0, 0)),
            pl.BlockSpec((1, f_out), lambda i, k: (0, 0)),
            pl.BlockSpec((_BR, n), lambda i, k: (i * _GROUP + k, 0)),
        ],
        out_specs=pl.BlockSpec((_BR * _GROUP, f_out), lambda i, k: (i, 0)),
        out_shape=jax.ShapeDtypeStruct((adj.shape[0], f_out), jnp.float32),
        scratch_shapes=[pltpu.VMEM((n, f_out), jnp.float32)],
    )(input, weight, bias2d, adj)


# speedup vs baseline: 1.0393x; 1.0005x over previous
import jax
import jax.numpy as jnp
from jax.experimental import pallas as pl
from jax.experimental.pallas import tpu as pltpu

_BR = 400


def _gc_kernel(x_ref, w_ref, b_ref, adj_ref, out_ref, support_ref):
    @pl.when(pl.program_id(0) == 0)
    def _():
        support_ref[...] = jnp.dot(
            x_ref[...], w_ref[...], preferred_element_type=jnp.float32
        )

    out_ref[...] = (
        jnp.dot(adj_ref[...], support_ref[...], preferred_element_type=jnp.float32)
        + b_ref[...]
    )


def kernel(input, adj, weight, bias):
    n, f_in = input.shape
    f_out = weight.shape[1]
    bias2d = bias.reshape(1, f_out)
    grid = (adj.shape[0] // _BR,)
    return pl.pallas_call(
        _gc_kernel,
        grid=grid,
        in_specs=[
            pl.BlockSpec((n, f_in), lambda i: (0, 0)),
            pl.BlockSpec((f_in, f_out), lambda i: (0, 0)),
            pl.BlockSpec((1, f_out), lambda i: (0, 0)),
            pl.BlockSpec((_BR, n), lambda i: (i, 0)),
        ],
        out_specs=pl.BlockSpec((_BR, f_out), lambda i: (i, 0)),
        out_shape=jax.ShapeDtypeStruct((adj.shape[0], f_out), jnp.float32),
        scratch_shapes=[pltpu.VMEM((n, f_out), jnp.float32)],
    )(input, weight, bias2d, adj)
